# baseline (speedup 1.0000x reference)
import functools

import jax
import jax.numpy as jnp
from jax import lax
from jax.experimental import pallas as pl
from jax.experimental.pallas import tpu as pltpu
from jax.experimental.pallas import tpu_sc as plsc

_N_VOCAB = 1000000
_BATCH = 16384
_NC = 2
_NS = 16
_NW = _NC * _NS
_PER_W = _BATCH // _NW
_CHUNK = 128
_NCHUNK = _PER_W // _CHUNK

_mesh = plsc.VectorSubcoreMesh(core_axis_name="c", subcore_axis_name="s")


@functools.partial(
    pl.kernel,
    mesh=_mesh,
    out_type=jax.ShapeDtypeStruct((_NW, _NCHUNK, _CHUNK), jnp.float32),
    scratch_types=[
        pltpu.VMEM((_NCHUNK, _CHUNK), jnp.int32),
        pltpu.VMEM((_NCHUNK, _CHUNK), jnp.float32),
        pltpu.SemaphoreType.DMA,
        pltpu.SemaphoreType.DMA,
    ],
)
def _gather_bias(values_hbm, table_hbm, out_hbm, idx_v, rows_v, sem, wsem):
    wid = lax.axis_index("s") * _NC + lax.axis_index("c")
    pltpu.sync_copy(values_hbm.at[wid], idx_v)
    copies = [
        pltpu.async_copy(table_hbm.at[idx_v.at[j]], rows_v.at[j], sem)
        for j in range(_NCHUNK)
    ]
    writes = []
    for j in range(_NCHUNK):
        copies[j].wait()
        writes.append(pltpu.async_copy(rows_v.at[j], out_hbm.at[wid, j], wsem))
    for w in writes:
        w.wait()


def kernel(values, data_bias_weight):
    table = data_bias_weight.reshape(_N_VOCAB)
    vals3 = values.astype(jnp.int32).reshape(_NW, _NCHUNK, _CHUNK)
    out3 = _gather_bias(vals3, table)
    return out3.reshape(_BATCH)


# speedup vs baseline: 1.0020x; 1.0020x over previous
import functools

import jax
import jax.numpy as jnp
from jax import lax
from jax.experimental import pallas as pl
from jax.experimental.pallas import tpu as pltpu
from jax.experimental.pallas import tpu_sc as plsc

_N_VOCAB = 1000000
_BATCH = 16384
_NC = 2
_NS = 16
_NW = _NC * _NS
_PER_W = _BATCH // _NW
_CHUNK = 128
_NCHUNK = _PER_W // _CHUNK

_mesh = plsc.VectorSubcoreMesh(core_axis_name="c", subcore_axis_name="s")


@functools.partial(
    pl.kernel,
    mesh=_mesh,
    out_type=jax.ShapeDtypeStruct((_NW, _NCHUNK, _CHUNK), jnp.float32),
    scratch_types=[
        pltpu.VMEM((_NCHUNK, _CHUNK), jnp.int32),
        pltpu.VMEM((_NCHUNK, _CHUNK), jnp.float32),
        pltpu.SemaphoreType.DMA,
        pltpu.SemaphoreType.DMA,
        pltpu.SemaphoreType.DMA,
    ],
)
def _gather_bias(values_hbm, table_hbm, out_hbm, idx_v, rows_v, isem, gsem, wsem):
    wid = lax.axis_index("s") * _NC + lax.axis_index("c")
    stages = [
        pltpu.async_copy(values_hbm.at[wid, j], idx_v.at[j], isem)
        for j in range(_NCHUNK)
    ]
    gathers = []
    for j in range(_NCHUNK):
        stages[j].wait()
        gathers.append(
            pltpu.async_copy(table_hbm.at[idx_v.at[j]], rows_v.at[j], gsem))
    writes = []
    for j in range(_NCHUNK):
        gathers[j].wait()
        writes.append(pltpu.async_copy(rows_v.at[j], out_hbm.at[wid, j], wsem))
    for w in writes:
        w.wait()


def kernel(values, data_bias_weight):
    table = data_bias_weight.reshape(_N_VOCAB)
    vals3 = values.astype(jnp.int32).reshape(_NW, _NCHUNK, _CHUNK)
    out3 = _gather_bias(vals3, table)
    return out3.reshape(_BATCH)
